# baseline (speedup 1.0000x reference)
import jax
import jax.numpy as jnp
from jax import lax
from jax.experimental import pallas as pl
from jax.experimental.pallas import tpu as pltpu
from jax.experimental.pallas import tpu_sc as plsc

V = 1000000
D = 32
B = 16384
L = 16
NC, NS = 2, 16
NW = NC * NS
BPW = B // NW


def _sc_kernel(w_hbm, n_hbm, idx_hbm, out_hbm, idx_v, buf_a, buf_b,
               sem_a, sem_b):
    wid = lax.axis_index("s") * NC + lax.axis_index("c")
    base = wid * BPW
    pltpu.sync_copy(idx_hbm.at[pl.ds(base, BPW)], idx_v)

    cps = []
    for j in range(D):
        cps.append(pltpu.async_copy(w_hbm.at[j].at[idx_v], buf_a.at[j], sem_a))
        cps.append(pltpu.async_copy(n_hbm.at[j].at[idx_v], buf_b.at[j], sem_b))
    for cp in cps:
        cp.wait()

    def add_body(c, carry):
        sl = pl.ds(c * L, L)
        for r in range(D):
            buf_a[r, sl] = buf_a[r, sl] + buf_b[r, sl]
        return carry

    lax.fori_loop(0, BPW // L, add_body, 0)
    pltpu.sync_copy(buf_a, out_hbm.at[:, pl.ds(base, BPW)])


@jax.jit
def _run(weight, name_emb, idx):
    fn = pl.kernel(
        _sc_kernel,
        out_type=jax.ShapeDtypeStruct((D, B), jnp.float32),
        mesh=plsc.VectorSubcoreMesh(core_axis_name="c", subcore_axis_name="s"),
        compiler_params=pltpu.CompilerParams(
            needs_layout_passes=False, use_tc_tiling_on_sc=False),
        scratch_types=[
            pltpu.VMEM((BPW,), jnp.int32),
            pltpu.VMEM((D, BPW), jnp.float32),
            pltpu.VMEM((D, BPW), jnp.float32),
            pltpu.SemaphoreType.DMA,
            pltpu.SemaphoreType.DMA,
        ],
    )
    return fn(weight.T, name_emb.T, idx).T


def kernel(weight, name_emb, idx):
    return _run(weight, name_emb, idx.astype(jnp.int32))


# speedup vs baseline: 4.8415x; 4.8415x over previous
import jax
import jax.numpy as jnp
from jax import lax
from jax.experimental import pallas as pl
from jax.experimental.pallas import tpu as pltpu
from jax.experimental.pallas import tpu_sc as plsc

D = 32
B = 16384
NC, NS = 2, 16
NW = NC * NS
BPW = B // NW


def _sc_kernel(w_hbm, n_hbm, idx_hbm, out_hbm, idx_v, rows_a, rows_b,
               sem_a, sem_b):
    wid = lax.axis_index("s") * NC + lax.axis_index("c")
    base = wid * BPW
    pltpu.sync_copy(idx_hbm.at[pl.ds(base, BPW)], idx_v)
    cp_a = pltpu.async_copy(w_hbm.at[idx_v], rows_a, sem_a)
    cp_b = pltpu.async_copy(n_hbm.at[idx_v], rows_b, sem_b)
    cp_a.wait()
    cp_b.wait()

    def body(r, carry):
        rows_a[r, :] = rows_a[r, :] + rows_b[r, :]
        return carry

    lax.fori_loop(0, BPW, body, 0)
    pltpu.sync_copy(rows_a, out_hbm.at[pl.ds(base, BPW)])


@jax.jit
def _run(weight, name_emb, idx):
    fn = pl.kernel(
        _sc_kernel,
        out_type=jax.ShapeDtypeStruct((B, D), jnp.bfloat16),
        mesh=plsc.VectorSubcoreMesh(core_axis_name="c", subcore_axis_name="s"),
        compiler_params=pltpu.CompilerParams(
            needs_layout_passes=False, use_tc_tiling_on_sc=False),
        scratch_types=[
            pltpu.VMEM((BPW,), jnp.int32),
            pltpu.VMEM((BPW, D), jnp.bfloat16),
            pltpu.VMEM((BPW, D), jnp.bfloat16),
            pltpu.SemaphoreType.DMA,
            pltpu.SemaphoreType.DMA,
        ],
    )
    out = fn(weight.astype(jnp.bfloat16), name_emb.astype(jnp.bfloat16), idx)
    return out.astype(jnp.float32)


def kernel(weight, name_emb, idx):
    return _run(weight, name_emb, idx.astype(jnp.int32))


# speedup vs baseline: 5.6389x; 1.1647x over previous
import functools

import jax
import jax.numpy as jnp
from jax import lax
from jax.experimental import pallas as pl
from jax.experimental.pallas import tpu as pltpu
from jax.experimental.pallas import tpu_sc as plsc

D = 32
B = 16384
L = 16
NC, NS = 2, 16
NW = NC * NS
BPW = B // NW


def _sc_kernel(w_hbm, n_hbm, idx_hbm, out_hbm, idx_v, rows_a, rows_b,
               sem_a, sem_b):
    wid = lax.axis_index("s") * NC + lax.axis_index("c")
    base = wid * BPW
    pltpu.sync_copy(idx_hbm.at[pl.ds(base, BPW)], idx_v)
    cp_a = pltpu.async_copy(w_hbm.at[idx_v], rows_a, sem_a)
    cp_b = pltpu.async_copy(n_hbm.at[idx_v], rows_b, sem_b)
    cp_a.wait()
    cp_b.wait()

    def body(r, carry):
        for c in range(D // L):
            sl = pl.ds(c * L, L)
            rows_a[r, sl] = rows_a[r, sl] + rows_b[r, sl]
        return carry

    lax.fori_loop(0, BPW, body, 0)
    pltpu.sync_copy(rows_a, out_hbm.at[pl.ds(base, BPW)])


@functools.partial(jax.jit, static_argnums=())
def _run(weight, name_emb, idx):
    fn = pl.kernel(
        _sc_kernel,
        out_type=jax.ShapeDtypeStruct((B, D), jnp.float32),
        mesh=plsc.VectorSubcoreMesh(core_axis_name="c", subcore_axis_name="s"),
        compiler_params=pltpu.CompilerParams(use_tc_tiling_on_sc=False),
        scratch_types=[
            pltpu.VMEM((BPW,), jnp.int32),
            pltpu.VMEM((BPW, D), jnp.float32),
            pltpu.VMEM((BPW, D), jnp.float32),
            pltpu.SemaphoreType.DMA,
            pltpu.SemaphoreType.DMA,
        ],
    )
    return fn(weight, name_emb, idx)


def kernel(weight, name_emb, idx):
    return _run(weight, name_emb, idx.astype(jnp.int32))
